# baseline (speedup 1.0000x reference)
import functools

import jax
import jax.numpy as jnp
from jax import lax
from jax.experimental import pallas as pl
from jax.experimental.pallas import tpu as pltpu
from jax.experimental.pallas import tpu_sc as plsc

_B = 16384
_D = 32
_NW = 32
_BPW = _B // _NW
_CH = 128
_NCH = _BPW // _CH
_GROUPS = _BPW // 16


def _body(uid_hbm, iid_hbm, uemb_hbm, iemb_hbm, out_hbm,
          uidx_v, iidx_v, urows_v, irows_v, out_v, sem):
    wid = lax.axis_index("s") * 2 + lax.axis_index("c")
    base = wid * _BPW

    copies = []
    for c in range(_NCH):
        pltpu.sync_copy(uid_hbm.at[pl.ds(base + c * _CH, _CH)], uidx_v.at[c])
        copies.append(
            pltpu.async_copy(uemb_hbm.at[uidx_v.at[c]],
                             urows_v.at[pl.ds(c * _CH, _CH)], sem))
    for c in range(_NCH):
        pltpu.sync_copy(iid_hbm.at[pl.ds(base + c * _CH, _CH)], iidx_v.at[c])
        copies.append(
            pltpu.async_copy(iemb_hbm.at[iidx_v.at[c]],
                             irows_v.at[pl.ds(c * _CH, _CH)], sem))
    for cp in copies:
        cp.wait()

    iota16 = lax.iota(jnp.int32, 16)

    def group(g, carry):
        rows = g * 16 + iota16
        acc = jnp.zeros((16,), jnp.float32)
        for f in range(_D):
            fv = jnp.full((16,), f, jnp.int32)
            u = plsc.load_gather(urows_v, [rows, fv])
            v = plsc.load_gather(irows_v, [rows, fv])
            acc = acc + u * v
        out_v[pl.ds(g * 16, 16)] = 1.0 / (1.0 + jnp.exp(-acc))
        return carry

    lax.fori_loop(0, _GROUPS, group, 0)
    pltpu.sync_copy(out_v, out_hbm.at[pl.ds(base, _BPW)])


@jax.jit
def _run(users_id, items_id, user_emb, item_emb):
    mesh = plsc.VectorSubcoreMesh(core_axis_name="c", subcore_axis_name="s")
    fn = functools.partial(
        pl.kernel,
        mesh=mesh,
        out_type=jax.ShapeDtypeStruct((_B,), jnp.float32),
        scratch_types=[
            pltpu.VMEM((_NCH, _CH), jnp.int32),
            pltpu.VMEM((_NCH, _CH), jnp.int32),
            pltpu.VMEM((_BPW, _D), jnp.float32),
            pltpu.VMEM((_BPW, _D), jnp.float32),
            pltpu.VMEM((_BPW,), jnp.float32),
            pltpu.SemaphoreType.DMA,
        ],
        compiler_params=pltpu.CompilerParams(
            needs_layout_passes=False, use_tc_tiling_on_sc=False),
    )(_body)
    return fn(users_id, items_id, user_emb, item_emb)


def kernel(users_id, items_id, user_emb, item_emb):
    return _run(users_id.astype(jnp.int32), items_id.astype(jnp.int32),
                user_emb, item_emb)
